# Initial kernel scaffold; baseline (speedup 1.0000x reference)
import functools

import jax
import jax.numpy as jnp
from jax import lax
from jax.experimental import pallas as pl
from jax.experimental.pallas import tpu as pltpu
from jax.experimental.pallas import tpu_sc as plsc

_NC, _NS = 2, 16
_NW = _NC * _NS
_LANE = 128
_CHUNK_ROWS = 8


@functools.lru_cache(maxsize=None)
def _build(n_idx_rows: int, vocab: int, dim: int):
    rows_per_w = n_idx_rows // _NW
    n_chunks = rows_per_w // _CHUNK_ROWS
    chunk_elems = _CHUNK_ROWS * _LANE

    mesh = plsc.VectorSubcoreMesh(core_axis_name="c", subcore_axis_name="s")

    @functools.partial(
        pl.kernel,
        out_type=jax.ShapeDtypeStruct((n_idx_rows * _LANE, dim), jnp.float32),
        mesh=mesh,
        scratch_types=[
            pltpu.VMEM((rows_per_w, _LANE), jnp.int32),
            pltpu.VMEM((chunk_elems, dim), jnp.float32),
            pltpu.SemaphoreType.DMA,
        ],
    )
    def gather_kernel(idx_hbm, table_hbm, out_hbm, idx_v, rows_v, gsem):
        wid = lax.axis_index("s") * _NC + lax.axis_index("c")
        row0 = wid * rows_per_w
        out0 = row0 * _LANE
        pltpu.sync_copy(idx_hbm.at[pl.ds(row0, rows_per_w)], idx_v)

        def chunk_body(c, carry):
            copies = []
            for j in range(_CHUNK_ROWS):
                copies.append(pltpu.async_copy(
                    table_hbm.at[idx_v.at[c * _CHUNK_ROWS + j]],
                    rows_v.at[pl.ds(j * _LANE, _LANE)],
                    gsem,
                ))
            for cp in copies:
                cp.wait()
            pltpu.sync_copy(
                rows_v, out_hbm.at[pl.ds(out0 + c * chunk_elems, chunk_elems)])
            return carry

        lax.fori_loop(0, n_chunks, chunk_body, 0)

    return gather_kernel


def kernel(inputs, weight):
    batch, hist = inputs.shape
    vocab, dim = weight.shape
    idx = inputs.reshape(-1, _LANE).astype(jnp.int32)
    fn = _build(idx.shape[0], vocab, dim)
    out = fn(idx, weight)
    return out.reshape(batch, hist, dim)


# speedup vs baseline: 1.4764x; 1.4764x over previous
import functools

import jax
import jax.numpy as jnp
from jax import lax
from jax.experimental import pallas as pl
from jax.experimental.pallas import tpu as pltpu
from jax.experimental.pallas import tpu_sc as plsc

_NC, _NS = 2, 16
_NW = _NC * _NS
_LANE = 128
_CHUNK_ROWS = 8


@functools.lru_cache(maxsize=None)
def _build(n_idx_rows: int, vocab: int, dim: int):
    rows_per_w = n_idx_rows // _NW
    n_chunks = rows_per_w // _CHUNK_ROWS
    chunk_elems = _CHUNK_ROWS * _LANE

    mesh = plsc.VectorSubcoreMesh(core_axis_name="c", subcore_axis_name="s")

    @functools.partial(
        pl.kernel,
        out_type=jax.ShapeDtypeStruct((n_idx_rows * _LANE, dim), jnp.float32),
        mesh=mesh,
        scratch_types=[
            pltpu.VMEM((rows_per_w, _LANE), jnp.int32),
            pltpu.VMEM((chunk_elems, dim), jnp.float32),
            pltpu.SemaphoreType.DMA,
        ],
        compiler_params=pltpu.CompilerParams(use_tc_tiling_on_sc=False),
    )
    def gather_kernel(idx_hbm, table_hbm, out_hbm, idx_v, rows_v, gsem):
        wid = lax.axis_index("s") * _NC + lax.axis_index("c")
        row0 = wid * rows_per_w
        out0 = row0 * _LANE
        pltpu.sync_copy(idx_hbm.at[pl.ds(row0, rows_per_w)], idx_v)

        def chunk_body(c, carry):
            copies = []
            for j in range(_CHUNK_ROWS):
                copies.append(pltpu.async_copy(
                    table_hbm.at[idx_v.at[c * _CHUNK_ROWS + j]],
                    rows_v.at[pl.ds(j * _LANE, _LANE)],
                    gsem,
                ))
            for cp in copies:
                cp.wait()
            pltpu.sync_copy(
                rows_v, out_hbm.at[pl.ds(out0 + c * chunk_elems, chunk_elems)])
            return carry

        lax.fori_loop(0, n_chunks, chunk_body, 0)

    return gather_kernel


def kernel(inputs, weight):
    batch, hist = inputs.shape
    vocab, dim = weight.shape
    idx = inputs.reshape(-1, _LANE).astype(jnp.int32)
    fn = _build(idx.shape[0], vocab, dim)
    out = fn(idx, weight)
    return out.reshape(batch, hist, dim)


# speedup vs baseline: 1.4827x; 1.0043x over previous
import functools

import jax
import jax.numpy as jnp
from jax import lax
from jax.experimental import pallas as pl
from jax.experimental.pallas import tpu as pltpu
from jax.experimental.pallas import tpu_sc as plsc

_NC, _NS = 2, 16
_NW = _NC * _NS
_CHUNK = 1280


@functools.lru_cache(maxsize=None)
def _build(n_idx: int, vocab: int, dim: int):
    idx_per_w = n_idx // _NW
    n_chunks = idx_per_w // _CHUNK

    mesh = plsc.VectorSubcoreMesh(core_axis_name="c", subcore_axis_name="s")

    @functools.partial(
        pl.kernel,
        out_type=jax.ShapeDtypeStruct((n_idx, dim), jnp.float32),
        mesh=mesh,
        scratch_types=[
            pltpu.VMEM((idx_per_w,), jnp.int32),
            pltpu.VMEM((_CHUNK, dim), jnp.float32),
            pltpu.SemaphoreType.DMA,
        ],
        compiler_params=pltpu.CompilerParams(use_tc_tiling_on_sc=False),
    )
    def gather_kernel(idx_hbm, table_hbm, out_hbm, idx_v, rows_v, gsem):
        wid = lax.axis_index("s") * _NC + lax.axis_index("c")
        base = wid * idx_per_w
        pltpu.sync_copy(idx_hbm.at[pl.ds(base, idx_per_w)], idx_v)

        def chunk_body(c, carry):
            pltpu.async_copy(
                table_hbm.at[idx_v.at[pl.ds(c * _CHUNK, _CHUNK)]],
                rows_v, gsem).wait()
            pltpu.sync_copy(
                rows_v, out_hbm.at[pl.ds(base + c * _CHUNK, _CHUNK)])
            return carry

        lax.fori_loop(0, n_chunks, chunk_body, 0)

    return gather_kernel


def kernel(inputs, weight):
    batch, hist = inputs.shape
    vocab, dim = weight.shape
    idx = inputs.reshape(-1).astype(jnp.int32)
    fn = _build(idx.shape[0], vocab, dim)
    out = fn(idx, weight)
    return out.reshape(batch, hist, dim)


# speedup vs baseline: 1.4920x; 1.0063x over previous
import functools

import jax
import jax.numpy as jnp
from jax import lax
from jax.experimental import pallas as pl
from jax.experimental.pallas import tpu as pltpu
from jax.experimental.pallas import tpu_sc as plsc

_NC, _NS = 2, 16
_NW = _NC * _NS
_CHUNK = 1280
_NBUF = 2


@functools.lru_cache(maxsize=None)
def _build(n_idx: int, vocab: int, dim: int):
    idx_per_w = n_idx // _NW
    n_chunks = idx_per_w // _CHUNK
    assert n_chunks % _NBUF == 0 and n_chunks >= 2 * _NBUF

    mesh = plsc.VectorSubcoreMesh(core_axis_name="c", subcore_axis_name="s")

    @functools.partial(
        pl.kernel,
        out_type=jax.ShapeDtypeStruct((n_idx, dim), jnp.float32),
        mesh=mesh,
        scratch_types=[
            pltpu.VMEM((idx_per_w,), jnp.int32),
            [pltpu.VMEM((_CHUNK, dim), jnp.float32) for _ in range(_NBUF)],
            [pltpu.SemaphoreType.DMA for _ in range(_NBUF)],
            [pltpu.SemaphoreType.DMA for _ in range(_NBUF)],
        ],
        compiler_params=pltpu.CompilerParams(use_tc_tiling_on_sc=False),
    )
    def gather_kernel(idx_hbm, table_hbm, out_hbm, idx_v, rows, gsems, wsems):
        wid = lax.axis_index("s") * _NC + lax.axis_index("c")
        base = wid * idx_per_w
        pltpu.sync_copy(idx_hbm.at[pl.ds(base, idx_per_w)], idx_v)

        def fire_gather(c, b):
            pltpu.async_copy(
                table_hbm.at[idx_v.at[pl.ds(c * _CHUNK, _CHUNK)]],
                rows[b], gsems[b])

        def wait_gather(b):
            pltpu.make_async_copy(
                table_hbm.at[pl.ds(0, _CHUNK)], rows[b], gsems[b]).wait()

        def fire_wb(c, b):
            pltpu.async_copy(
                rows[b], out_hbm.at[pl.ds(base + c * _CHUNK, _CHUNK)],
                wsems[b])

        def wait_wb(b):
            pltpu.make_async_copy(
                rows[b], out_hbm.at[pl.ds(base, _CHUNK)], wsems[b]).wait()

        for b in range(_NBUF):
            fire_gather(b, b)

        def outer(i, carry):
            c0 = i * _NBUF
            for b in range(_NBUF):
                wait_gather(b)
                fire_wb(c0 + b, b)
            for b in range(_NBUF):
                wait_wb(b)
                fire_gather(c0 + _NBUF + b, b)
            return carry

        lax.fori_loop(0, n_chunks // _NBUF - 1, outer, 0)

        c0 = n_chunks - _NBUF
        for b in range(_NBUF):
            wait_gather(b)
            fire_wb(c0 + b, b)
        for b in range(_NBUF):
            wait_wb(b)

    return gather_kernel


def kernel(inputs, weight):
    batch, hist = inputs.shape
    vocab, dim = weight.shape
    idx = inputs.reshape(-1).astype(jnp.int32)
    fn = _build(idx.shape[0], vocab, dim)
    out = fn(idx, weight)
    return out.reshape(batch, hist, dim)


# speedup vs baseline: 1.4921x; 1.0000x over previous
import functools

import jax
import jax.numpy as jnp
from jax import lax
from jax.experimental import pallas as pl
from jax.experimental.pallas import tpu as pltpu
from jax.experimental.pallas import tpu_sc as plsc

_NC, _NS = 2, 16
_NW = _NC * _NS
_CHUNK = 1280
_NBUF = 2


@functools.lru_cache(maxsize=None)
def _build(n_idx: int, vocab: int, dim: int):
    idx_per_w = n_idx // _NW
    n_chunks = idx_per_w // _CHUNK
    assert n_chunks % _NBUF == 0 and n_chunks >= 2 * _NBUF

    mesh = plsc.VectorSubcoreMesh(core_axis_name="c", subcore_axis_name="s")

    @functools.partial(
        pl.kernel,
        out_type=jax.ShapeDtypeStruct((n_idx, dim), jnp.float32),
        mesh=mesh,
        scratch_types=[
            pltpu.VMEM((idx_per_w,), jnp.int32),
            [pltpu.VMEM((_CHUNK, dim), jnp.float32) for _ in range(_NBUF)],
            [pltpu.SemaphoreType.DMA for _ in range(_NBUF)],
            [pltpu.SemaphoreType.DMA for _ in range(_NBUF)],
        ],
        compiler_params=pltpu.CompilerParams(use_tc_tiling_on_sc=False),
    )
    def gather_kernel(idx_hbm, table_hbm, out_hbm, idx_v, rows, gsems, wsems):
        wid = lax.axis_index("s") * _NC + lax.axis_index("c")
        base = wid * idx_per_w
        pltpu.sync_copy(idx_hbm.at[pl.ds(base, idx_per_w)], idx_v)

        def fire_gather(c, b):
            pltpu.async_copy(
                table_hbm.at[idx_v.at[pl.ds(c * _CHUNK, _CHUNK)]],
                rows[b], gsems[b])

        def wait_gather(b):
            pltpu.make_async_copy(
                table_hbm.at[idx_v.at[pl.ds(0, _CHUNK)]],
                rows[b], gsems[b]).wait()

        def fire_wb(c, b):
            pltpu.async_copy(
                rows[b], out_hbm.at[pl.ds(base + c * _CHUNK, _CHUNK)],
                wsems[b])

        def wait_wb(b):
            pltpu.make_async_copy(
                rows[b], out_hbm.at[pl.ds(base, _CHUNK)], wsems[b]).wait()

        for b in range(_NBUF):
            fire_gather(b, b)

        def outer(i, carry):
            c0 = i * _NBUF
            for b in range(_NBUF):
                wait_gather(b)
                fire_wb(c0 + b, b)
            for b in range(_NBUF):
                wait_wb(b)
                fire_gather(c0 + _NBUF + b, b)
            return carry

        lax.fori_loop(0, n_chunks // _NBUF - 1, outer, 0)

        c0 = n_chunks - _NBUF
        for b in range(_NBUF):
            wait_gather(b)
            fire_wb(c0 + b, b)
        for b in range(_NBUF):
            wait_wb(b)

    return gather_kernel


def kernel(inputs, weight):
    batch, hist = inputs.shape
    vocab, dim = weight.shape
    idx = inputs.reshape(-1).astype(jnp.int32)
    fn = _build(idx.shape[0], vocab, dim)
    out = fn(idx, weight)
    return out.reshape(batch, hist, dim)
